# Initial kernel scaffold; baseline (speedup 1.0000x reference)
import functools

import jax
import jax.numpy as jnp
from jax import lax
from jax.experimental import pallas as pl
from jax.experimental.pallas import tpu as pltpu
from jax.experimental.pallas import tpu_sc as plsc

Bb, C, N, K = 4, 64, 2048, 16
CF = 4 * C
NP = 128
NBLK = N // NP
EPS = 1e-5

NC, NS = 2, 16
NW = NC * NS
ROWS = Bb * N * K
RPW = ROWS // NW
CH = 128
NCH = RPW // CH


def _stage_a_body(x_ref, win_ref, bin_ref, g1_ref, be1_ref, wq_ref, wk_ref,
                  wv_ref, h_ref, q_ref, kv_ref):
    xb = x_ref[0]
    h = jnp.dot(xb, win_ref[...].T, preferred_element_type=jnp.float32)
    h = h + bin_ref[...]
    mu = jnp.mean(h, axis=-1, keepdims=True)
    var = jnp.mean((h - mu) ** 2, axis=-1, keepdims=True)
    hn = (h - mu) / jnp.sqrt(var + EPS) * g1_ref[...] + be1_ref[...]
    q = jnp.dot(hn, wq_ref[...].T, preferred_element_type=jnp.float32)
    k = jnp.dot(hn, wk_ref[...].T, preferred_element_type=jnp.float32)
    v = jnp.dot(hn, wv_ref[...].T, preferred_element_type=jnp.float32)
    h_ref[0] = h
    q_ref[0] = q
    kv_ref[0, :, :C] = k
    kv_ref[0, :, C:] = v


def _stage_a(xT, W_in, b_in, g1, be1, W_q, W_k, W_v):
    blk = 512
    grid = (Bb, N // blk)
    full = lambda shape: pl.BlockSpec(shape, lambda b, i: (0, 0))
    return pl.pallas_call(
        _stage_a_body,
        grid=grid,
        in_specs=[
            pl.BlockSpec((1, blk, C), lambda b, i: (b, i, 0)),
            full((C, C)), full((1, C)), full((1, C)), full((1, C)),
            full((C, C)), full((C, C)), full((C, C)),
        ],
        out_specs=[
            pl.BlockSpec((1, blk, C), lambda b, i: (b, i, 0)),
            pl.BlockSpec((1, blk, C), lambda b, i: (b, i, 0)),
            pl.BlockSpec((1, blk, 2 * C), lambda b, i: (b, i, 0)),
        ],
        out_shape=[
            jax.ShapeDtypeStruct((Bb, N, C), jnp.float32),
            jax.ShapeDtypeStruct((Bb, N, C), jnp.float32),
            jax.ShapeDtypeStruct((Bb, N, 2 * C), jnp.float32),
        ],
    )(xT, W_in, b_in.reshape(1, C), g1.reshape(1, C), be1.reshape(1, C),
      W_q, W_k, W_v)


def _stage_k_body(prow_ref, pcol_ref, idx_ref):
    b = pl.program_id(0)
    pcol = pcol_ref[0]
    nj = jnp.sum(pcol * pcol, axis=0, keepdims=True)
    rblk = 128

    def blk_body(i, _):
        pb = prow_ref[0, pl.ds(i * rblk, rblk), :]
        ni = jnp.sum(pb * pb, axis=1, keepdims=True)
        dist = 2.0 * jnp.dot(pb, pcol, preferred_element_type=jnp.float32)
        dist = dist - ni - nj
        col = lax.broadcasted_iota(jnp.int32, (rblk, N), 1)
        row = i * rblk + lax.broadcasted_iota(jnp.int32, (rblk, N), 0)
        dist = jnp.where(col == row, -1e9, dist)
        cols = []
        for _t in range(K):
            m = jnp.max(dist, axis=1, keepdims=True)
            cand = jnp.min(jnp.where(dist >= m, col, N), axis=1,
                           keepdims=True)
            cols.append(cand)
            dist = jnp.where(col == cand, -3e38, dist)
        idxblk = jnp.concatenate(cols, axis=1)
        idx_ref[0, pl.ds(i * rblk, rblk), :] = idxblk + b * N
        return 0

    lax.fori_loop(0, N // rblk, blk_body, 0)


def _stage_k(pT16, p16N):
    return pl.pallas_call(
        _stage_k_body,
        grid=(Bb,),
        in_specs=[
            pl.BlockSpec((1, N, 16), lambda b: (b, 0, 0)),
            pl.BlockSpec((1, 16, N), lambda b: (b, 0, 0)),
        ],
        out_specs=pl.BlockSpec((1, N, K), lambda b: (b, 0, 0)),
        out_shape=jax.ShapeDtypeStruct((Bb, N, K), jnp.int32),
    )(pT16, p16N)


def _sc_gather_body(idx_hbm, kv_hbm, p_hbm, kvj_hbm, pj_hbm,
                    idx_v, kv_v, p_v, sem1, sem2):
    wid = lax.axis_index("s") * NC + lax.axis_index("c")
    base = wid * RPW

    def chunk(i, _):
        off = base + i * CH
        pltpu.sync_copy(idx_hbm.at[pl.ds(off, CH)], idx_v)
        cp1 = pltpu.async_copy(kv_hbm.at[idx_v], kv_v, sem1)
        cp2 = pltpu.async_copy(p_hbm.at[idx_v], p_v, sem2)
        cp1.wait()
        cp2.wait()
        pltpu.sync_copy(kv_v, kvj_hbm.at[pl.ds(off, CH)])
        pltpu.sync_copy(p_v, pj_hbm.at[pl.ds(off, CH)])
        return 0

    lax.fori_loop(0, NCH, chunk, 0)


def _stage_g(flat_idx, kv_tab, p_tab):
    mesh = plsc.VectorSubcoreMesh(core_axis_name="c", subcore_axis_name="s")
    fn = pl.kernel(
        _sc_gather_body,
        out_type=[
            jax.ShapeDtypeStruct((ROWS, 2 * C), jnp.float32),
            jax.ShapeDtypeStruct((ROWS, 16), jnp.float32),
        ],
        mesh=mesh,
        scratch_types=[
            pltpu.VMEM((CH,), jnp.int32),
            pltpu.VMEM((CH, 2 * C), jnp.float32),
            pltpu.VMEM((CH, 16), jnp.float32),
            pltpu.SemaphoreType.DMA,
            pltpu.SemaphoreType.DMA,
        ],
    )
    return fn(flat_idx, kv_tab, p_tab)


def _stage_c_body(h_ref, q_ref, pi_ref, kvj_ref, pj_ref, wd1_ref, bd1_ref,
                  wd2_ref, bd2_ref, wa_ref, ba_ref, g2_ref, be2_ref,
                  wf1_ref, bf1_ref, wf2_ref, bf2_ref, out_ref):
    R = K * NP
    kv = kvj_ref[0].reshape(R, 2 * C)
    pj = pj_ref[0].reshape(R, 16)
    pi = jnp.broadcast_to(pi_ref[0][None], (K, NP, 16)).reshape(R, 16)
    dp = pi - pj
    r1 = jnp.dot(dp, wd1_ref[...].T, preferred_element_type=jnp.float32)
    r1 = jnp.maximum(r1 + bd1_ref[...], 0.0)
    d = jnp.dot(r1, wd2_ref[...].T, preferred_element_type=jnp.float32)
    d = d + bd2_ref[...]
    qi = jnp.broadcast_to(q_ref[0][None], (K, NP, C)).reshape(R, C)
    t = qi - kv[:, :C] + d
    logits = jnp.dot(t, wa_ref[...].T, preferred_element_type=jnp.float32)
    logits = (logits + ba_ref[...]).reshape(K, NP, C)
    m = jnp.max(logits, axis=0, keepdims=True)
    e = jnp.exp(logits - m)
    s = jnp.sum(e, axis=0, keepdims=True)
    a = e / s
    vpd = (kv[:, C:] + d).reshape(K, NP, C)
    y = jnp.sum(a * vpd, axis=0)
    h1 = h_ref[0] + y
    mu = jnp.mean(h1, axis=-1, keepdims=True)
    var = jnp.mean((h1 - mu) ** 2, axis=-1, keepdims=True)
    h2 = (h1 - mu) / jnp.sqrt(var + EPS) * g2_ref[...] + be2_ref[...]
    f = jnp.dot(h2, wf1_ref[...].T, preferred_element_type=jnp.float32)
    f = jnp.maximum(f + bf1_ref[...], 0.0)
    ff = jnp.dot(f, wf2_ref[...].T, preferred_element_type=jnp.float32)
    out_ref[0] = h1 + ff + bf2_ref[...]


def _stage_c(h, q, pT16, kvj, pj, W_d1p, b_d1, W_d2, b_d2, W_a, b_a,
             g2, be2, W_f1, b_f1, W_f2, b_f2):
    grid = (Bb, NBLK)

    def full(shape):
        return pl.BlockSpec(shape, lambda b, i: tuple(0 for _ in shape))

    return pl.pallas_call(
        _stage_c_body,
        grid=grid,
        in_specs=[
            pl.BlockSpec((1, NP, C), lambda b, i: (b, i, 0)),
            pl.BlockSpec((1, NP, C), lambda b, i: (b, i, 0)),
            pl.BlockSpec((1, NP, 16), lambda b, i: (b, i, 0)),
            pl.BlockSpec((1, K, NP, 2 * C), lambda b, i: (b, 0, i, 0)),
            pl.BlockSpec((1, K, NP, 16), lambda b, i: (b, 0, i, 0)),
            full((C, 16)), full((1, C)), full((C, C)), full((1, C)),
            full((C, C)), full((1, C)), full((1, C)), full((1, C)),
            full((CF, C)), full((1, CF)), full((C, CF)), full((1, C)),
        ],
        out_specs=pl.BlockSpec((1, NP, C), lambda b, i: (b, i, 0)),
        out_shape=jax.ShapeDtypeStruct((Bb, N, C), jnp.float32),
    )(h, q, pT16, kvj, pj,
      W_d1p, b_d1.reshape(1, C), W_d2, b_d2.reshape(1, C), W_a,
      b_a.reshape(1, C), g2.reshape(1, C), be2.reshape(1, C),
      W_f1, b_f1.reshape(1, CF), W_f2, b_f2.reshape(1, C))


def kernel(x, p, W_in, b_in, W_q, W_k, W_v, W_d1, b_d1, W_d2, b_d2, W_a,
           b_a, g1, be1, g2, be2, W_f1, b_f1, W_f2, b_f2):
    xT = jnp.transpose(x, (0, 2, 1))
    pT16 = jnp.pad(jnp.transpose(p, (0, 2, 1)), ((0, 0), (0, 0), (0, 13)))
    p16N = jnp.pad(p, ((0, 0), (0, 13), (0, 0)))
    W_d1p = jnp.pad(W_d1, ((0, 0), (0, 13)))

    h, q, kv = _stage_a(xT, W_in, b_in, g1, be1, W_q, W_k, W_v)
    idx = _stage_k(pT16, p16N)

    flat_idx = jnp.transpose(idx, (0, 2, 1)).reshape(ROWS)
    kvj, pj = _stage_g(flat_idx, kv.reshape(Bb * N, 2 * C),
                       pT16.reshape(Bb * N, 16))
    kvj = kvj.reshape(Bb, K, N, 2 * C)
    pj = pj.reshape(Bb, K, N, 16)

    out = _stage_c(h, q, pT16, kvj, pj, W_d1p, b_d1, W_d2, b_d2, W_a, b_a,
                   g2, be2, W_f1, b_f1, W_f2, b_f2)
    return jnp.transpose(out, (0, 2, 1))


# speedup vs baseline: 6.7735x; 6.7735x over previous
import jax
import jax.numpy as jnp
from jax import lax
from jax.experimental import pallas as pl
from jax.experimental.pallas import tpu as pltpu
from jax.experimental.pallas import tpu_sc as plsc

Bb, C, N, K = 4, 64, 2048, 16
CF = 4 * C
W = 4 * C
NP = 128
NBLK = N // NP
EPS = 1e-5

NC, NS = 2, 16
NW = NC * NS
ROWS = Bb * N * K
RPW = ROWS // NW
CH = 128
NCH = RPW // CH


def _stage_a_body(x_ref, p_ref, win_ref, bin_ref, g1_ref, be1_ref, wq_ref,
                  wk_ref, wv_ref, wd1_ref, h_ref, kvs_ref):
    xb = x_ref[0]
    h = jnp.dot(xb, win_ref[...].T, preferred_element_type=jnp.float32)
    h = h + bin_ref[...]
    mu = jnp.mean(h, axis=-1, keepdims=True)
    var = jnp.mean((h - mu) ** 2, axis=-1, keepdims=True)
    hn = (h - mu) / jnp.sqrt(var + EPS) * g1_ref[...] + be1_ref[...]
    k = jnp.dot(hn, wk_ref[...].T, preferred_element_type=jnp.float32)
    v = jnp.dot(hn, wv_ref[...].T, preferred_element_type=jnp.float32)
    s = jnp.dot(p_ref[0], wd1_ref[...].T, preferred_element_type=jnp.float32)
    q = jnp.dot(hn, wq_ref[...].T, preferred_element_type=jnp.float32)
    h_ref[0] = h
    kvs_ref[0, :, 0 * C:1 * C] = k
    kvs_ref[0, :, 1 * C:2 * C] = v
    kvs_ref[0, :, 2 * C:3 * C] = s
    kvs_ref[0, :, 3 * C:4 * C] = q


def _stage_a(xT, pT16, W_in, b_in, g1, be1, W_q, W_k, W_v, W_d1p):
    blk = 512
    grid = (Bb, N // blk)
    full = lambda shape: pl.BlockSpec(shape, lambda b, i: (0, 0))
    return pl.pallas_call(
        _stage_a_body,
        grid=grid,
        in_specs=[
            pl.BlockSpec((1, blk, C), lambda b, i: (b, i, 0)),
            pl.BlockSpec((1, blk, 16), lambda b, i: (b, i, 0)),
            full((C, C)), full((1, C)), full((1, C)), full((1, C)),
            full((C, C)), full((C, C)), full((C, C)), full((C, 16)),
        ],
        out_specs=[
            pl.BlockSpec((1, blk, C), lambda b, i: (b, i, 0)),
            pl.BlockSpec((1, blk, W), lambda b, i: (b, i, 0)),
        ],
        out_shape=[
            jax.ShapeDtypeStruct((Bb, N, C), jnp.float32),
            jax.ShapeDtypeStruct((Bb, N, W), jnp.float32),
        ],
    )(xT, pT16, W_in, b_in.reshape(1, C), g1.reshape(1, C),
      be1.reshape(1, C), W_q, W_k, W_v, W_d1p)


def _stage_k_body(prow_ref, pcol_ref, idx_ref):
    b = pl.program_id(0)
    pcol = pcol_ref[0]
    nj = jnp.sum(pcol * pcol, axis=0, keepdims=True)
    rblk = 128

    def blk_body(i, _):
        pb = prow_ref[0, pl.ds(i * rblk, rblk), :]
        ni = jnp.sum(pb * pb, axis=1, keepdims=True)
        dist = 2.0 * jnp.dot(pb, pcol, preferred_element_type=jnp.float32)
        dist = dist - ni - nj
        col = lax.broadcasted_iota(jnp.int32, (rblk, N), 1)
        row = i * rblk + lax.broadcasted_iota(jnp.int32, (rblk, N), 0)
        dist = jnp.where(col == row, -1e9, dist)
        cols = []
        for _t in range(K):
            m = jnp.max(dist, axis=1, keepdims=True)
            cand = jnp.min(jnp.where(dist >= m, col, N), axis=1,
                           keepdims=True)
            cols.append(cand)
            dist = jnp.where(col == cand, -3e38, dist)
        idxblk = jnp.concatenate(cols, axis=1)
        idx_ref[0, pl.ds(i * rblk, rblk), :] = idxblk + b * N
        return 0

    lax.fori_loop(0, N // rblk, blk_body, 0)


def _stage_k(pT16, p16N):
    return pl.pallas_call(
        _stage_k_body,
        grid=(Bb,),
        in_specs=[
            pl.BlockSpec((1, N, 16), lambda b: (b, 0, 0)),
            pl.BlockSpec((1, 16, N), lambda b: (b, 0, 0)),
        ],
        out_specs=pl.BlockSpec((1, N, K), lambda b: (b, 0, 0)),
        out_shape=jax.ShapeDtypeStruct((Bb, N, K), jnp.int32),
    )(pT16, p16N)


def _sc_gather_body(idx_hbm, kvs_hbm, out_hbm, idx_v, row_v, sem):
    wid = lax.axis_index("s") * NC + lax.axis_index("c")
    base = wid * RPW

    def chunk(i, _):
        off = base + i * CH
        pltpu.sync_copy(idx_hbm.at[pl.ds(off, CH)], idx_v)
        pltpu.async_copy(kvs_hbm.at[idx_v], row_v, sem).wait()
        pltpu.sync_copy(row_v, out_hbm.at[pl.ds(off, CH)])
        return 0

    lax.fori_loop(0, NCH, chunk, 0)


def _stage_g(flat_idx, kvs_tab):
    mesh = plsc.VectorSubcoreMesh(core_axis_name="c", subcore_axis_name="s")
    fn = pl.kernel(
        _sc_gather_body,
        out_type=jax.ShapeDtypeStruct((ROWS, W), jnp.float32),
        mesh=mesh,
        scratch_types=[
            pltpu.VMEM((CH,), jnp.int32),
            pltpu.VMEM((CH, W), jnp.float32),
            pltpu.SemaphoreType.DMA,
        ],
    )
    return fn(flat_idx, kvs_tab)


def _stage_c_body(h_ref, kvsi_ref, kvsj_ref, bd1_ref, wd2_ref, bd2_ref,
                  wa_ref, ba_ref, g2_ref, be2_ref, wf1_ref, bf1_ref,
                  wf2_ref, bf2_ref, out_ref):
    R = K * NP
    kvs_i = kvsi_ref[0]
    kvs_j = kvsj_ref[0].reshape(R, W)
    s_i = jnp.broadcast_to(kvs_i[None, :, 2 * C:3 * C],
                           (K, NP, C)).reshape(R, C)
    q_i = jnp.broadcast_to(kvs_i[None, :, 3 * C:4 * C],
                           (K, NP, C)).reshape(R, C)
    r1 = jnp.maximum(s_i - kvs_j[:, 2 * C:3 * C] + bd1_ref[...], 0.0)
    d = jnp.dot(r1, wd2_ref[...].T, preferred_element_type=jnp.float32)
    d = d + bd2_ref[...]
    t = q_i - kvs_j[:, :C] + d
    logits = jnp.dot(t, wa_ref[...].T, preferred_element_type=jnp.float32)
    logits = (logits + ba_ref[...]).reshape(K, NP, C)
    m = jnp.max(logits, axis=0, keepdims=True)
    e = jnp.exp(logits - m)
    s = jnp.sum(e, axis=0, keepdims=True)
    a = e / s
    vpd = (kvs_j[:, C:2 * C] + d).reshape(K, NP, C)
    y = jnp.sum(a * vpd, axis=0)
    h1 = h_ref[0] + y
    mu = jnp.mean(h1, axis=-1, keepdims=True)
    var = jnp.mean((h1 - mu) ** 2, axis=-1, keepdims=True)
    h2 = (h1 - mu) / jnp.sqrt(var + EPS) * g2_ref[...] + be2_ref[...]
    f = jnp.dot(h2, wf1_ref[...].T, preferred_element_type=jnp.float32)
    f = jnp.maximum(f + bf1_ref[...], 0.0)
    ff = jnp.dot(f, wf2_ref[...].T, preferred_element_type=jnp.float32)
    out_ref[0] = h1 + ff + bf2_ref[...]


def _stage_c(h, kvs, kvsj, b_d1, W_d2, b_d2, W_a, b_a, g2, be2,
             W_f1, b_f1, W_f2, b_f2):
    grid = (Bb, NBLK)

    def full(shape):
        return pl.BlockSpec(shape, lambda b, i: tuple(0 for _ in shape))

    return pl.pallas_call(
        _stage_c_body,
        grid=grid,
        in_specs=[
            pl.BlockSpec((1, NP, C), lambda b, i: (b, i, 0)),
            pl.BlockSpec((1, NP, W), lambda b, i: (b, i, 0)),
            pl.BlockSpec((1, K, NP, W), lambda b, i: (b, 0, i, 0)),
            full((1, C)), full((C, C)), full((1, C)),
            full((C, C)), full((1, C)), full((1, C)), full((1, C)),
            full((CF, C)), full((1, CF)), full((C, CF)), full((1, C)),
        ],
        out_specs=pl.BlockSpec((1, NP, C), lambda b, i: (b, i, 0)),
        out_shape=jax.ShapeDtypeStruct((Bb, N, C), jnp.float32),
    )(h, kvs, kvsj,
      b_d1.reshape(1, C), W_d2, b_d2.reshape(1, C), W_a,
      b_a.reshape(1, C), g2.reshape(1, C), be2.reshape(1, C),
      W_f1, b_f1.reshape(1, CF), W_f2, b_f2.reshape(1, C))


def kernel(x, p, W_in, b_in, W_q, W_k, W_v, W_d1, b_d1, W_d2, b_d2, W_a,
           b_a, g1, be1, g2, be2, W_f1, b_f1, W_f2, b_f2):
    xT = jnp.transpose(x, (0, 2, 1))
    pT16 = jnp.pad(jnp.transpose(p, (0, 2, 1)), ((0, 0), (0, 0), (0, 13)))
    p16N = jnp.pad(p, ((0, 0), (0, 13), (0, 0)))
    W_d1p = jnp.pad(W_d1, ((0, 0), (0, 13)))

    h, kvs = _stage_a(xT, pT16, W_in, b_in, g1, be1, W_q, W_k, W_v, W_d1p)
    idx = _stage_k(pT16, p16N)

    flat_idx = jnp.transpose(idx, (0, 2, 1)).reshape(ROWS)
    kvsj = _stage_g(flat_idx, kvs.reshape(Bb * N, W))
    kvsj = kvsj.reshape(Bb, K, N, W)

    out = _stage_c(h, kvs, kvsj, b_d1, W_d2, b_d2, W_a, b_a,
                   g2, be2, W_f1, b_f1, W_f2, b_f2)
    return jnp.transpose(out, (0, 2, 1))


# speedup vs baseline: 9.3365x; 1.3784x over previous
import jax
import jax.numpy as jnp
from jax import lax
from jax.experimental import pallas as pl
from jax.experimental.pallas import tpu as pltpu
from jax.experimental.pallas import tpu_sc as plsc

Bb, C, N, K = 4, 64, 2048, 16
CF = 4 * C
W = 2 * C
NP = 128
NBLK = N // NP
EPS = 1e-5

NC, NS = 2, 16
NW = NC * NS
ROWS = Bb * N * K
RPW = ROWS // NW
CH = 128
NCH = RPW // CH


def _pack_bf16(x):
    b = lax.bitcast_convert_type(x, jnp.int32)
    bh = (b + jnp.int32(0x7FFF) + ((b >> 16) & 1)) & jnp.int32(-65536)
    lo = (bh[:, :C // 2] >> 16) & jnp.int32(0xFFFF)
    hi = bh[:, C // 2:]
    return lo | hi


def _unpack_bf16(w):
    lo = lax.bitcast_convert_type(w << 16, jnp.float32)
    hi = lax.bitcast_convert_type(w & jnp.int32(-65536), jnp.float32)
    return jnp.concatenate([lo, hi], axis=1)


def _stage_a_body(x_ref, p_ref, win_ref, bin_ref, g1_ref, be1_ref, wq_ref,
                  wk_ref, wv_ref, wd1_ref, h_ref, kvs_ref):
    xb = x_ref[0]
    h = jnp.dot(xb, win_ref[...].T, preferred_element_type=jnp.float32)
    h = h + bin_ref[...]
    mu = jnp.mean(h, axis=-1, keepdims=True)
    var = jnp.mean((h - mu) ** 2, axis=-1, keepdims=True)
    hn = (h - mu) / jnp.sqrt(var + EPS) * g1_ref[...] + be1_ref[...]
    k = jnp.dot(hn, wk_ref[...].T, preferred_element_type=jnp.float32)
    v = jnp.dot(hn, wv_ref[...].T, preferred_element_type=jnp.float32)
    s = jnp.dot(p_ref[0], wd1_ref[...].T, preferred_element_type=jnp.float32)
    q = jnp.dot(hn, wq_ref[...].T, preferred_element_type=jnp.float32)
    h_ref[0] = h
    hw = C // 2
    kvs_ref[0, :, 0 * hw:1 * hw] = _pack_bf16(k)
    kvs_ref[0, :, 1 * hw:2 * hw] = _pack_bf16(v)
    kvs_ref[0, :, 2 * hw:3 * hw] = _pack_bf16(s)
    kvs_ref[0, :, 3 * hw:4 * hw] = _pack_bf16(q)


def _stage_a(xT, pT16, W_in, b_in, g1, be1, W_q, W_k, W_v, W_d1p):
    blk = 512
    grid = (Bb, N // blk)
    full = lambda shape: pl.BlockSpec(shape, lambda b, i: (0, 0))
    return pl.pallas_call(
        _stage_a_body,
        grid=grid,
        in_specs=[
            pl.BlockSpec((1, blk, C), lambda b, i: (b, i, 0)),
            pl.BlockSpec((1, blk, 16), lambda b, i: (b, i, 0)),
            full((C, C)), full((1, C)), full((1, C)), full((1, C)),
            full((C, C)), full((C, C)), full((C, C)), full((C, 16)),
        ],
        out_specs=[
            pl.BlockSpec((1, blk, C), lambda b, i: (b, i, 0)),
            pl.BlockSpec((1, blk, W), lambda b, i: (b, i, 0)),
        ],
        out_shape=[
            jax.ShapeDtypeStruct((Bb, N, C), jnp.float32),
            jax.ShapeDtypeStruct((Bb, N, W), jnp.int32),
        ],
    )(xT, pT16, W_in, b_in.reshape(1, C), g1.reshape(1, C),
      be1.reshape(1, C), W_q, W_k, W_v, W_d1p)


def _stage_k_body(prow_ref, pcol_ref, idx_ref):
    b = pl.program_id(0)
    pcol = pcol_ref[0]
    nj = jnp.sum(pcol * pcol, axis=0, keepdims=True)
    rblk = 128

    def blk_body(i, _):
        pb = prow_ref[0, pl.ds(i * rblk, rblk), :]
        ni = jnp.sum(pb * pb, axis=1, keepdims=True)
        dist = 2.0 * jnp.dot(pb, pcol, preferred_element_type=jnp.float32)
        dist = dist - ni - nj
        col = lax.broadcasted_iota(jnp.int32, (rblk, N), 1)
        row = i * rblk + lax.broadcasted_iota(jnp.int32, (rblk, N), 0)
        dist = jnp.where(col == row, -1e9, dist)
        bits = lax.bitcast_convert_type(jnp.minimum(dist, -1e-30),
                                        jnp.int32)
        u = (~bits & jnp.int32(-2048)) | (jnp.int32(N - 1) - col)
        cols = []
        for _t in range(K):
            m = jnp.max(u, axis=1, keepdims=True)
            cols.append(jnp.int32(N - 1) - (m & jnp.int32(N - 1)))
            u = jnp.where(u == m, 0, u)
        idxblk = jnp.concatenate(cols, axis=1)
        idx_ref[0, pl.ds(i * rblk, rblk), :] = idxblk + b * N
        return 0

    lax.fori_loop(0, N // rblk, blk_body, 0)


def _stage_k(pT16, p16N):
    return pl.pallas_call(
        _stage_k_body,
        grid=(Bb,),
        in_specs=[
            pl.BlockSpec((1, N, 16), lambda b: (b, 0, 0)),
            pl.BlockSpec((1, 16, N), lambda b: (b, 0, 0)),
        ],
        out_specs=pl.BlockSpec((1, N, K), lambda b: (b, 0, 0)),
        out_shape=jax.ShapeDtypeStruct((Bb, N, K), jnp.int32),
    )(pT16, p16N)


def _sc_gather_body(idx_hbm, kvs_hbm, out_hbm, idx0, idx1, row0, row1,
                    sem0, sem1):
    wid = lax.axis_index("s") * NC + lax.axis_index("c")
    base = wid * RPW
    bufs = ((idx0, row0, sem0), (idx1, row1, sem1))

    def fire(i):
        idx_v, row_v, sem = bufs[i % 2]
        pltpu.sync_copy(idx_hbm.at[pl.ds(base + i * CH, CH)], idx_v)
        return pltpu.async_copy(kvs_hbm.at[idx_v], row_v, sem)

    cp = fire(0)
    for i in range(NCH):
        nxt = fire(i + 1) if i + 1 < NCH else None
        cp.wait()
        row_v = bufs[i % 2][1]
        pltpu.sync_copy(row_v, out_hbm.at[pl.ds(base + i * CH, CH)])
        cp = nxt


def _stage_g(flat_idx, kvs_tab):
    mesh = plsc.VectorSubcoreMesh(core_axis_name="c", subcore_axis_name="s")
    fn = pl.kernel(
        _sc_gather_body,
        out_type=jax.ShapeDtypeStruct((ROWS, W), jnp.int32),
        mesh=mesh,
        scratch_types=[
            pltpu.VMEM((CH,), jnp.int32),
            pltpu.VMEM((CH,), jnp.int32),
            pltpu.VMEM((CH, W), jnp.int32),
            pltpu.VMEM((CH, W), jnp.int32),
            pltpu.SemaphoreType.DMA,
            pltpu.SemaphoreType.DMA,
        ],
    )
    return fn(flat_idx, kvs_tab)


def _stage_c_body(h_ref, kvsi_ref, kvsj_ref, bd1_ref, wd2_ref, bd2_ref,
                  wa_ref, ba_ref, g2_ref, be2_ref, wf1_ref, bf1_ref,
                  wf2_ref, bf2_ref, out_ref):
    R = K * NP
    hw = C // 2
    kvs_i = kvsi_ref[0]
    kvs_j = kvsj_ref[0].reshape(R, W)
    s_iu = _unpack_bf16(kvs_i[:, 2 * hw:3 * hw])
    q_iu = _unpack_bf16(kvs_i[:, 3 * hw:4 * hw])
    s_i = jnp.broadcast_to(s_iu[None], (K, NP, C)).reshape(R, C)
    q_i = jnp.broadcast_to(q_iu[None], (K, NP, C)).reshape(R, C)
    k_j = _unpack_bf16(kvs_j[:, 0 * hw:1 * hw])
    v_j = _unpack_bf16(kvs_j[:, 1 * hw:2 * hw])
    s_j = _unpack_bf16(kvs_j[:, 2 * hw:3 * hw])
    r1 = jnp.maximum(s_i - s_j + bd1_ref[...], 0.0)
    d = jnp.dot(r1, wd2_ref[...].T, preferred_element_type=jnp.float32)
    d = d + bd2_ref[...]
    t = q_i - k_j + d
    logits = jnp.dot(t, wa_ref[...].T, preferred_element_type=jnp.float32)
    logits = (logits + ba_ref[...]).reshape(K, NP, C)
    m = jnp.max(logits, axis=0, keepdims=True)
    e = jnp.exp(logits - m)
    s = jnp.sum(e, axis=0, keepdims=True)
    a = e / s
    vpd = (v_j + d).reshape(K, NP, C)
    y = jnp.sum(a * vpd, axis=0)
    h1 = h_ref[0] + y
    mu = jnp.mean(h1, axis=-1, keepdims=True)
    var = jnp.mean((h1 - mu) ** 2, axis=-1, keepdims=True)
    h2 = (h1 - mu) / jnp.sqrt(var + EPS) * g2_ref[...] + be2_ref[...]
    f = jnp.dot(h2, wf1_ref[...].T, preferred_element_type=jnp.float32)
    f = jnp.maximum(f + bf1_ref[...], 0.0)
    ff = jnp.dot(f, wf2_ref[...].T, preferred_element_type=jnp.float32)
    out_ref[0] = h1 + ff + bf2_ref[...]


def _stage_c(h, kvs, kvsj, b_d1, W_d2, b_d2, W_a, b_a, g2, be2,
             W_f1, b_f1, W_f2, b_f2):
    grid = (Bb, NBLK)

    def full(shape):
        return pl.BlockSpec(shape, lambda b, i: tuple(0 for _ in shape))

    return pl.pallas_call(
        _stage_c_body,
        grid=grid,
        in_specs=[
            pl.BlockSpec((1, NP, C), lambda b, i: (b, i, 0)),
            pl.BlockSpec((1, NP, W), lambda b, i: (b, i, 0)),
            pl.BlockSpec((1, K, NP, W), lambda b, i: (b, 0, i, 0)),
            full((1, C)), full((C, C)), full((1, C)),
            full((C, C)), full((1, C)), full((1, C)), full((1, C)),
            full((CF, C)), full((1, CF)), full((C, CF)), full((1, C)),
        ],
        out_specs=pl.BlockSpec((1, NP, C), lambda b, i: (b, i, 0)),
        out_shape=jax.ShapeDtypeStruct((Bb, N, C), jnp.float32),
    )(h, kvs, kvsj,
      b_d1.reshape(1, C), W_d2, b_d2.reshape(1, C), W_a,
      b_a.reshape(1, C), g2.reshape(1, C), be2.reshape(1, C),
      W_f1, b_f1.reshape(1, CF), W_f2, b_f2.reshape(1, C))


def kernel(x, p, W_in, b_in, W_q, W_k, W_v, W_d1, b_d1, W_d2, b_d2, W_a,
           b_a, g1, be1, g2, be2, W_f1, b_f1, W_f2, b_f2):
    xT = jnp.transpose(x, (0, 2, 1))
    pT16 = jnp.pad(jnp.transpose(p, (0, 2, 1)), ((0, 0), (0, 0), (0, 13)))
    p16N = jnp.pad(p, ((0, 0), (0, 13), (0, 0)))
    W_d1p = jnp.pad(W_d1, ((0, 0), (0, 13)))

    h, kvs = _stage_a(xT, pT16, W_in, b_in, g1, be1, W_q, W_k, W_v, W_d1p)
    idx = _stage_k(pT16, p16N)

    flat_idx = jnp.transpose(idx, (0, 2, 1)).reshape(ROWS)
    kvsj = _stage_g(flat_idx, kvs.reshape(Bb * N, W))
    kvsj = kvsj.reshape(Bb, K, N, W)

    out = _stage_c(h, kvs, kvsj, b_d1, W_d2, b_d2, W_a, b_a,
                   g2, be2, W_f1, b_f1, W_f2, b_f2)
    return jnp.transpose(out, (0, 2, 1))
